# baseline (speedup 1.0000x reference)
import functools

import jax
import jax.numpy as jnp
from jax import lax
from jax.experimental import pallas as pl
from jax.experimental.pallas import tpu as pltpu
from jax.experimental.pallas import tpu_sc as plsc

EMB_DIM = 64
BATCH = 16384

NUM_CORES = 2
NUM_SUBCORES = 16
LANES = 16
NW = NUM_CORES * NUM_SUBCORES
B_PER_W = BATCH // NW
CHUNK = 128
NCHUNK = B_PER_W // CHUNK
GROUPS = CHUNK // LANES
VROW = 2 * EMB_DIM

_mesh = plsc.VectorSubcoreMesh(
    core_axis_name="c", subcore_axis_name="s",
    num_cores=NUM_CORES, num_subcores=NUM_SUBCORES,
)


@functools.partial(
    pl.kernel,
    out_type=jax.ShapeDtypeStruct((BATCH,), jnp.float32),
    mesh=_mesh,
    scratch_types=[
        pltpu.VMEM((NCHUNK, CHUNK), jnp.int32),
        pltpu.VMEM((NCHUNK, CHUNK), jnp.int32),
        pltpu.VMEM((NCHUNK, CHUNK), jnp.float32),
        pltpu.VMEM((NCHUNK, CHUNK), jnp.float32),
        pltpu.SemaphoreType.DMA,
    ],
    compiler_params=pltpu.CompilerParams(
        needs_layout_passes=False, use_tc_tiling_on_sc=False),
)
def _bias_sc(x1_hbm, x2_hbm, ubias_hbm, cbias_hbm, out_hbm,
             idx1_v, idx2_v, b1_v, b2_v, sem):
    wid = lax.axis_index("s") * NUM_CORES + lax.axis_index("c")
    base = wid * B_PER_W

    for j in range(NCHUNK):
        pltpu.sync_copy(x1_hbm.at[pl.ds(base + j * CHUNK, CHUNK)], idx1_v.at[j])
        pltpu.sync_copy(x2_hbm.at[pl.ds(base + j * CHUNK, CHUNK)], idx2_v.at[j])

    copies = []
    for j in range(NCHUNK):
        copies.append(pltpu.async_copy(ubias_hbm.at[idx1_v.at[j]], b1_v.at[j], sem))
        copies.append(pltpu.async_copy(cbias_hbm.at[idx2_v.at[j]], b2_v.at[j], sem))
    for cp in copies:
        cp.wait()

    for j in range(NCHUNK):
        @pl.loop(0, GROUPS)
        def _(g):
            col = pl.ds(g * LANES, LANES)
            b1_v[j, col] = b1_v[j, col] + b2_v[j, col]

        pltpu.sync_copy(b1_v.at[j], out_hbm.at[pl.ds(base + j * CHUNK, CHUNK)])


@functools.partial(
    pl.kernel,
    out_type=jax.ShapeDtypeStruct((BATCH,), jnp.float32),
    mesh=_mesh,
    scratch_types=[
        pltpu.VMEM((NCHUNK, CHUNK), jnp.int32),
        pltpu.VMEM((NCHUNK, CHUNK), jnp.int32),
        pltpu.VMEM((NCHUNK, CHUNK), jnp.int32),
        pltpu.VMEM((NCHUNK, CHUNK), jnp.int32),
        pltpu.VMEM((2, CHUNK, VROW), jnp.float32),
        pltpu.VMEM((2, CHUNK, VROW), jnp.float32),
        pltpu.VMEM((NCHUNK, CHUNK), jnp.float32),
        pltpu.VMEM((B_PER_W,), jnp.float32),
        pltpu.VMEM((LANES, LANES + 1), jnp.float32),
        pltpu.SemaphoreType.DMA,
        pltpu.SemaphoreType.DMA,
    ],
    compiler_params=pltpu.CompilerParams(
        needs_layout_passes=False, use_tc_tiling_on_sc=True),
)
def _ranker_sc(x1_hbm, x2_hbm, up_hbm, cp_hbm, bsum_hbm,
               out_hbm, idx1_v, idx2_v, idx1p_v, idx2p_v, u_v, c_v, bs_v,
               out_v, pad_v, sem0, sem1):
    wid = lax.axis_index("s") * NUM_CORES + lax.axis_index("c")
    base = wid * B_PER_W
    sems = [sem0, sem1]

    for j in range(NCHUNK):
        pltpu.sync_copy(x1_hbm.at[pl.ds(base + j * CHUNK, CHUNK)], idx1_v.at[j])
        pltpu.sync_copy(x2_hbm.at[pl.ds(base + j * CHUNK, CHUNK)], idx2_v.at[j])
        pltpu.sync_copy(bsum_hbm.at[pl.ds(base + j * CHUNK, CHUNK)], bs_v.at[j])

        @pl.loop(0, GROUPS)
        def _(g):
            col = pl.ds(g * LANES, LANES)
            v1 = idx1_v[j, col]
            v2 = idx2_v[j, col]
            idx1p_v[j, col] = lax.shift_left(
                lax.shift_right_logical(v1, LOG2_BLKL), LOG2_PACK) + (
                    v1 & (PACK - 1))
            idx2p_v[j, col] = lax.shift_left(
                lax.shift_right_logical(v2, LOG2_BLKL), LOG2_PACK) + (
                    v2 & (PACK - 1))

    def issue(j, buf, sem):
        pltpu.async_copy(up_hbm.at[idx1p_v.at[j]], u_v.at[buf], sem)
        pltpu.async_copy(cp_hbm.at[idx2p_v.at[j]], c_v.at[buf], sem)

    def drain(j, buf, sem):
        pltpu.make_async_copy(
            up_hbm.at[pl.ds(0, CHUNK), pl.ds(0, VROW)], u_v.at[buf], sem).wait()
        pltpu.make_async_copy(
            cp_hbm.at[pl.ds(0, CHUNK), pl.ds(0, VROW)], c_v.at[buf], sem).wait()

    iota16 = lax.iota(jnp.int32, LANES)

    def compute(j, buf):
        @pl.loop(0, GROUPS)
        def _(g):
            rbase = g * LANES
            par1 = (lax.shift_right_logical(
                idx1_v[j, pl.ds(rbase, LANES)], LOG2_PACK) & 1) * EMB_DIM
            par2 = (lax.shift_right_logical(
                idx2_v[j, pl.ds(rbase, LANES)], LOG2_PACK) & 1) * EMB_DIM
            for i in range(LANES):
                r = rbase + i
                uoff = par1[i]
                coff = par2[i]
                acc = (u_v[buf, r, pl.ds(uoff, LANES)]
                       * c_v[buf, r, pl.ds(coff, LANES)])
                for k in range(1, EMB_DIM // LANES):
                    acc = acc + (u_v[buf, r, pl.ds(uoff + k * LANES, LANES)]
                                 * c_v[buf, r, pl.ds(coff + k * LANES, LANES)])
                plsc.store_scatter(
                    pad_v, [iota16, jnp.full((LANES,), i, jnp.int32)], acc)
            dots = pad_v[0, pl.ds(0, LANES)]
            for l in range(1, LANES):
                dots = dots + pad_v[l, pl.ds(0, LANES)]
            col = pl.ds(rbase, LANES)
            tot = dots + float(EMB_DIM) * bs_v[j, col]
            out_v[pl.ds(j * CHUNK + rbase, LANES)] = 1.0 / (1.0 + jnp.exp(-tot))

    issue(0, 0, sems[0])
    for j in range(NCHUNK):
        if j + 1 < NCHUNK:
            issue(j + 1, (j + 1) % 2, sems[(j + 1) % 2])
        drain(j, j % 2, sems[j % 2])
        compute(j, j % 2)

    pltpu.sync_copy(out_v, out_hbm.at[pl.ds(base, B_PER_W)])


N_U = 1000000
N_C = 100000
PACK = 16384
LOG2_PACK = 14
BLKL = 2 * PACK
LOG2_BLKL = LOG2_PACK + 1
NBLK_U = (N_U + BLKL - 1) // BLKL
NBLK_C = (N_C + BLKL - 1) // BLKL


def _repack_body(t_ref, o_ref):
    o_ref[:, :EMB_DIM] = t_ref[:, :PACK].T
    o_ref[:, EMB_DIM:] = t_ref[:, PACK:].T


def _repack_tc(table_t, nblk):
    return pl.pallas_call(
        _repack_body,
        grid=(nblk,),
        in_specs=[pl.BlockSpec((EMB_DIM, BLKL), lambda j: (0, j))],
        out_specs=pl.BlockSpec((PACK, VROW), lambda j: (j, 0)),
        out_shape=jax.ShapeDtypeStruct((nblk * PACK, VROW), jnp.float32),
        compiler_params=pltpu.CompilerParams(
            dimension_semantics=("parallel",)),
    )(table_t)


def kernel(x1, x2, uemb, cemb, user_bias, creator_bias):
    x1 = x1.astype(jnp.int32)
    x2 = x2.astype(jnp.int32)
    bsum = _bias_sc(x1, x2, user_bias.T.reshape(-1), creator_bias.T.reshape(-1))
    up = _repack_tc(uemb.T, NBLK_U)
    cp = _repack_tc(cemb.T, NBLK_C)
    return _ranker_sc(x1, x2, up, cp, bsum)
